# baseline (speedup 1.0000x reference)
import jax
import jax.numpy as jnp
import numpy as np
from jax.experimental import pallas as pl
from jax.experimental.pallas import tpu as pltpu

_N = 10000
_BR1 = 200
_NB1 = _N // _BR1
_BR2 = 1000
_NR2 = _N // _BR2
_CW = 1280
_NC2 = -(-_N // _CW)
_UPPER = [(r, c) for r in range(_NR2) for c in range((r * _BR2) // _CW, _NC2)]
_T2 = len(_UPPER)


def _pass1_kernel(x_ref, a_ref, W1_ref, b1_ref,
                  z1_out_ref, z2p_out_ref,
                  z0_s, z1_s):
    r = pl.program_id(0)

    @pl.when(r == 0)
    def _init():
        z0_s[...] = jnp.maximum(
            jnp.dot(x_ref[...], W1_ref[...], preferred_element_type=jnp.float32)
            + b1_ref[...], 0.0)

    zb = jnp.dot(a_ref[...], z0_s[...], preferred_element_type=jnp.float32)
    z1_s[pl.ds(r * _BR1, _BR1), :] = zb
    z1_out_ref[...] = zb

    cutoff = (r * _BR1) // _BR2 * _BR2 // _CW * _CW
    rowids = jax.lax.broadcasted_iota(jnp.int32, (_N, 1), 0)
    zm = jnp.where(rowids < cutoff, z1_s[...], 0.0)
    z2p_out_ref[...] = jnp.dot(a_ref[...], zm, preferred_element_type=jnp.float32)


def _pass2_kernel(rows_ref, cols_ref,
                  a_ref, z1_ref, z2p_ref, gamma_ref, beta_ref,
                  Wp1_ref, bp1_ref, Wp2_ref, bp2_ref,
                  zn_ref, p_ref,
                  z2_s):
    t = pl.program_id(0)
    r = rows_ref[t]
    c = cols_ref[t]

    local_r = jax.lax.broadcasted_iota(jnp.int32, (_CW, 1), 0)
    z1m = jnp.where(local_r + c * _CW < _N, z1_ref[...], 0.0)
    local_c = jax.lax.broadcasted_iota(jnp.int32, (1, _CW), 1)
    am = jnp.where(local_c + c * _CW < _N, a_ref[...], 0.0)
    contrib = jnp.dot(am, z1m, preferred_element_type=jnp.float32)

    first = c == (r * _BR2) // _CW

    @pl.when(first)
    def _first():
        z2_s[pl.ds(r * _BR2, _BR2), :] = (
            z2p_ref[pl.ds(r * _BR2, _BR2), :] + contrib)

    @pl.when(jnp.logical_not(first))
    def _acc():
        z2_s[pl.ds(r * _BR2, _BR2), :] = (
            z2_s[pl.ds(r * _BR2, _BR2), :] + contrib)

    @pl.when(t == _T2 - 1)
    def _finish():
        z2 = z2_s[...]
        mean = jnp.mean(z2, axis=0, keepdims=True)
        var = jnp.mean((z2 - mean) ** 2, axis=0, keepdims=True)
        zn = (z2 - mean) * jax.lax.rsqrt(var + 1e-5) * gamma_ref[...] + beta_ref[...]
        zn_ref[...] = zn
        h = jnp.maximum(
            jnp.dot(zn, Wp1_ref[...], preferred_element_type=jnp.float32)
            + bp1_ref[...], 0.0)
        p_ref[...] = jnp.dot(
            h, Wp2_ref[...], preferred_element_type=jnp.float32) + bp2_ref[...]


def kernel(x, a, W1, b1, gamma, beta, Wp1, bp1, Wp2, bp2):
    emb = W1.shape[1]
    proj = Wp1.shape[1]

    z1, z2p = pl.pallas_call(
        _pass1_kernel,
        grid=(_NB1,),
        in_specs=[
            pl.BlockSpec(x.shape, lambda r: (0, 0)),
            pl.BlockSpec((_BR1, _N), lambda r: (r, 0)),
            pl.BlockSpec(W1.shape, lambda r: (0, 0)),
            pl.BlockSpec((1, emb), lambda r: (0, 0)),
        ],
        out_specs=[pl.BlockSpec((_BR1, emb), lambda r: (r, 0)),
                   pl.BlockSpec((_BR1, emb), lambda r: (r, 0))],
        out_shape=[jax.ShapeDtypeStruct((_N, emb), jnp.float32),
                   jax.ShapeDtypeStruct((_N, emb), jnp.float32)],
        scratch_shapes=[pltpu.VMEM((_N, emb), jnp.float32),
                        pltpu.VMEM((_N, emb), jnp.float32)],
    )(x, a, W1, b1.reshape(1, -1))

    rows = jnp.asarray(np.array([rc[0] for rc in _UPPER], dtype=np.int32))
    cols = jnp.asarray(np.array([rc[1] for rc in _UPPER], dtype=np.int32))

    def const2(shape):
        return pl.BlockSpec(shape, lambda t, rows, cols: (0, 0))

    zn, p = pl.pallas_call(
        _pass2_kernel,
        grid_spec=pltpu.PrefetchScalarGridSpec(
            num_scalar_prefetch=2,
            grid=(_T2,),
            in_specs=[
                pl.BlockSpec((_BR2, _CW), lambda t, rows, cols: (rows[t], cols[t])),
                pl.BlockSpec((_CW, emb), lambda t, rows, cols: (cols[t], 0)),
                const2((_N, emb)), const2((1, emb)), const2((1, emb)),
                const2((emb, proj)), const2((1, proj)),
                const2((proj, proj)), const2((1, proj)),
            ],
            out_specs=[const2((_N, emb)), const2((_N, proj))],
            scratch_shapes=[pltpu.VMEM((_N, emb), jnp.float32)],
        ),
        out_shape=[jax.ShapeDtypeStruct((_N, emb), jnp.float32),
                   jax.ShapeDtypeStruct((_N, proj), jnp.float32)],
    )(rows, cols, a, z1, z2p, gamma.reshape(1, -1), beta.reshape(1, -1),
      Wp1, bp1.reshape(1, -1), Wp2, bp2.reshape(1, -1))
    return (zn, p)


# speedup vs baseline: 1.2737x; 1.2737x over previous
import jax
import jax.numpy as jnp
import numpy as np
from jax.experimental import pallas as pl
from jax.experimental.pallas import tpu as pltpu

_N = 10000
_BR1 = 200
_NB1 = _N // _BR1
_BR2 = 1000
_NR2 = _N // _BR2
_CW = 1280
_NC2 = -(-_N // _CW)
_UPPER = [(r, c) for r in range(_NR2) for c in range((r * _BR2) // _CW, _NC2)]
_T2 = len(_UPPER)


def _pass1_kernel(x_ref, a_ref, W1_ref, b1_ref,
                  z1_out_ref, z2p_out_ref,
                  z0b_s, z1_s):
    r = pl.program_id(0)

    @pl.when(r == 0)
    def _init():
        z0 = jnp.maximum(
            jnp.dot(x_ref[...], W1_ref[...], preferred_element_type=jnp.float32)
            + b1_ref[...], 0.0)
        z0b_s[...] = z0.astype(jnp.bfloat16)

    ab = a_ref[...].astype(jnp.bfloat16)
    zb = jnp.dot(ab, z0b_s[...], preferred_element_type=jnp.float32)
    z1_s[pl.ds(r * _BR1, _BR1), :] = zb
    z1_out_ref[...] = zb

    cutoff = (r * _BR1) // _BR2 * _BR2 // _CW * _CW
    rowids = jax.lax.broadcasted_iota(jnp.int32, (_N, 1), 0)
    zm = jnp.where(rowids < cutoff, z1_s[...], 0.0).astype(jnp.bfloat16)
    z2p_out_ref[...] = jnp.dot(ab, zm, preferred_element_type=jnp.float32)


def _pass2_kernel(rows_ref, cols_ref,
                  a_ref, z1_ref, z2p_ref, gamma_ref, beta_ref,
                  Wp1_ref, bp1_ref, Wp2_ref, bp2_ref,
                  zn_ref, p_ref,
                  z2_s):
    t = pl.program_id(0)
    r = rows_ref[t]
    c = cols_ref[t]

    local_r = jax.lax.broadcasted_iota(jnp.int32, (_CW, 1), 0)
    z1m = jnp.where(local_r + c * _CW < _N, z1_ref[...], 0.0).astype(jnp.bfloat16)
    local_c = jax.lax.broadcasted_iota(jnp.int32, (1, _CW), 1)
    am = jnp.where(local_c + c * _CW < _N, a_ref[...], 0.0).astype(jnp.bfloat16)
    contrib = jnp.dot(am, z1m, preferred_element_type=jnp.float32)

    first = c == (r * _BR2) // _CW

    @pl.when(first)
    def _first():
        z2_s[pl.ds(r * _BR2, _BR2), :] = (
            z2p_ref[pl.ds(r * _BR2, _BR2), :] + contrib)

    @pl.when(jnp.logical_not(first))
    def _acc():
        z2_s[pl.ds(r * _BR2, _BR2), :] = (
            z2_s[pl.ds(r * _BR2, _BR2), :] + contrib)

    @pl.when(t == _T2 - 1)
    def _finish():
        z2 = z2_s[...]
        mean = jnp.mean(z2, axis=0, keepdims=True)
        var = jnp.mean((z2 - mean) ** 2, axis=0, keepdims=True)
        zn = (z2 - mean) * jax.lax.rsqrt(var + 1e-5) * gamma_ref[...] + beta_ref[...]
        zn_ref[...] = zn
        h = jnp.maximum(
            jnp.dot(zn, Wp1_ref[...], preferred_element_type=jnp.float32)
            + bp1_ref[...], 0.0)
        p_ref[...] = jnp.dot(
            h, Wp2_ref[...], preferred_element_type=jnp.float32) + bp2_ref[...]


def kernel(x, a, W1, b1, gamma, beta, Wp1, bp1, Wp2, bp2):
    emb = W1.shape[1]
    proj = Wp1.shape[1]

    z1, z2p = pl.pallas_call(
        _pass1_kernel,
        grid=(_NB1,),
        in_specs=[
            pl.BlockSpec(x.shape, lambda r: (0, 0)),
            pl.BlockSpec((_BR1, _N), lambda r: (r, 0)),
            pl.BlockSpec(W1.shape, lambda r: (0, 0)),
            pl.BlockSpec((1, emb), lambda r: (0, 0)),
        ],
        out_specs=[pl.BlockSpec((_BR1, emb), lambda r: (r, 0)),
                   pl.BlockSpec((_BR1, emb), lambda r: (r, 0))],
        out_shape=[jax.ShapeDtypeStruct((_N, emb), jnp.float32),
                   jax.ShapeDtypeStruct((_N, emb), jnp.float32)],
        scratch_shapes=[pltpu.VMEM((_N, emb), jnp.bfloat16),
                        pltpu.VMEM((_N, emb), jnp.float32)],
    )(x, a, W1, b1.reshape(1, -1))

    rows = jnp.asarray(np.array([rc[0] for rc in _UPPER], dtype=np.int32))
    cols = jnp.asarray(np.array([rc[1] for rc in _UPPER], dtype=np.int32))

    def const2(shape):
        return pl.BlockSpec(shape, lambda t, rows, cols: (0, 0))

    zn, p = pl.pallas_call(
        _pass2_kernel,
        grid_spec=pltpu.PrefetchScalarGridSpec(
            num_scalar_prefetch=2,
            grid=(_T2,),
            in_specs=[
                pl.BlockSpec((_BR2, _CW), lambda t, rows, cols: (rows[t], cols[t])),
                pl.BlockSpec((_CW, emb), lambda t, rows, cols: (cols[t], 0)),
                const2((_N, emb)), const2((1, emb)), const2((1, emb)),
                const2((emb, proj)), const2((1, proj)),
                const2((proj, proj)), const2((1, proj)),
            ],
            out_specs=[const2((_N, emb)), const2((_N, proj))],
            scratch_shapes=[pltpu.VMEM((_N, emb), jnp.float32)],
        ),
        out_shape=[jax.ShapeDtypeStruct((_N, emb), jnp.float32),
                   jax.ShapeDtypeStruct((_N, proj), jnp.float32)],
    )(rows, cols, a, z1, z2p, gamma.reshape(1, -1), beta.reshape(1, -1),
      Wp1, bp1.reshape(1, -1), Wp2, bp2.reshape(1, -1))
    return (zn, p)


# speedup vs baseline: 1.3086x; 1.0274x over previous
import jax
import jax.numpy as jnp
import numpy as np
from jax.experimental import pallas as pl
from jax.experimental.pallas import tpu as pltpu

_N = 10000
_BR1 = 200
_NB1 = _N // _BR1
_BR2 = 1000
_NR2 = _N // _BR2
_CW = 1280
_NC2 = -(-_N // _CW)
_UPPER = [(r, c) for r in range(_NR2) for c in range((r * _BR2) // _CW, _NC2)]
_T2 = len(_UPPER)
_NCH = (_NR2 - 1) * _BR2 // _CW


def _pass1_kernel(x_ref, a_ref, W1_ref, b1_ref,
                  z1_out_ref, z2p_out_ref,
                  z0b_s, z1b_s, ab_s):
    r = pl.program_id(0)
    emb = z0b_s.shape[1]

    @pl.when(r == 0)
    def _init():
        z0 = jnp.maximum(
            jnp.dot(x_ref[...], W1_ref[...], preferred_element_type=jnp.float32)
            + b1_ref[...], 0.0)
        z0b_s[...] = z0.astype(jnp.bfloat16)

    ab = a_ref[...].astype(jnp.bfloat16)
    zb = jnp.dot(ab, z0b_s[...], preferred_element_type=jnp.float32)
    zbb = zb.astype(jnp.bfloat16)
    z1b_s[pl.ds(r * _BR1, _BR1), :] = zbb
    z1_out_ref[...] = zbb

    for j in range(_NCH):
        ab_s[j] = ab[:, j * _CW:(j + 1) * _CW]

    nchunk = (r * _BR1) // _BR2 * _BR2 // _CW

    def _body(k, acc):
        z_c = z1b_s[pl.ds(k * _CW, _CW), :]
        return acc + jnp.dot(ab_s[k], z_c, preferred_element_type=jnp.float32)

    z2p_out_ref[...] = jax.lax.fori_loop(
        0, nchunk, _body, jnp.zeros((_BR1, emb), jnp.float32))


def _pass2_kernel(rows_ref, cols_ref,
                  a_ref, z1_ref, z2p_ref, gamma_ref, beta_ref,
                  Wp1_ref, bp1_ref, Wp2_ref, bp2_ref,
                  zn_ref, p_ref,
                  z2_s):
    t = pl.program_id(0)
    r = rows_ref[t]
    c = cols_ref[t]

    local_r = jax.lax.broadcasted_iota(jnp.int32, (_CW, 1), 0)
    z1m = jnp.where(local_r + c * _CW < _N, z1_ref[...], jnp.bfloat16(0))
    local_c = jax.lax.broadcasted_iota(jnp.int32, (1, _CW), 1)
    am = jnp.where(local_c + c * _CW < _N, a_ref[...], 0.0).astype(jnp.bfloat16)
    contrib = jnp.dot(am, z1m, preferred_element_type=jnp.float32)

    first = c == (r * _BR2) // _CW

    @pl.when(first)
    def _first():
        z2_s[pl.ds(r * _BR2, _BR2), :] = (
            z2p_ref[pl.ds(r * _BR2, _BR2), :] + contrib)

    @pl.when(jnp.logical_not(first))
    def _acc():
        z2_s[pl.ds(r * _BR2, _BR2), :] = (
            z2_s[pl.ds(r * _BR2, _BR2), :] + contrib)

    @pl.when(t == _T2 - 1)
    def _finish():
        z2 = z2_s[...]
        mean = jnp.mean(z2, axis=0, keepdims=True)
        var = jnp.mean((z2 - mean) ** 2, axis=0, keepdims=True)
        zn = (z2 - mean) * jax.lax.rsqrt(var + 1e-5) * gamma_ref[...] + beta_ref[...]
        zn_ref[...] = zn
        h = jnp.maximum(
            jnp.dot(zn, Wp1_ref[...], preferred_element_type=jnp.float32)
            + bp1_ref[...], 0.0)
        p_ref[...] = jnp.dot(
            h, Wp2_ref[...], preferred_element_type=jnp.float32) + bp2_ref[...]


def kernel(x, a, W1, b1, gamma, beta, Wp1, bp1, Wp2, bp2):
    emb = W1.shape[1]
    proj = Wp1.shape[1]

    z1, z2p = pl.pallas_call(
        _pass1_kernel,
        grid=(_NB1,),
        in_specs=[
            pl.BlockSpec(x.shape, lambda r: (0, 0)),
            pl.BlockSpec((_BR1, _N), lambda r: (r, 0)),
            pl.BlockSpec(W1.shape, lambda r: (0, 0)),
            pl.BlockSpec((1, emb), lambda r: (0, 0)),
        ],
        out_specs=[pl.BlockSpec((_BR1, emb), lambda r: (r, 0)),
                   pl.BlockSpec((_BR1, emb), lambda r: (r, 0))],
        out_shape=[jax.ShapeDtypeStruct((_N, emb), jnp.bfloat16),
                   jax.ShapeDtypeStruct((_N, emb), jnp.float32)],
        scratch_shapes=[pltpu.VMEM((_N, emb), jnp.bfloat16),
                        pltpu.VMEM((_N, emb), jnp.bfloat16),
                        pltpu.VMEM((_NCH, _BR1, _CW), jnp.bfloat16)],
    )(x, a, W1, b1.reshape(1, -1))

    rows = jnp.asarray(np.array([rc[0] for rc in _UPPER], dtype=np.int32))
    cols = jnp.asarray(np.array([rc[1] for rc in _UPPER], dtype=np.int32))

    def const2(shape):
        return pl.BlockSpec(shape, lambda t, rows, cols: (0, 0))

    zn, p = pl.pallas_call(
        _pass2_kernel,
        grid_spec=pltpu.PrefetchScalarGridSpec(
            num_scalar_prefetch=2,
            grid=(_T2,),
            in_specs=[
                pl.BlockSpec((_BR2, _CW), lambda t, rows, cols: (rows[t], cols[t])),
                pl.BlockSpec((_CW, emb), lambda t, rows, cols: (cols[t], 0)),
                const2((_N, emb)), const2((1, emb)), const2((1, emb)),
                const2((emb, proj)), const2((1, proj)),
                const2((proj, proj)), const2((1, proj)),
            ],
            out_specs=[const2((_N, emb)), const2((_N, proj))],
            scratch_shapes=[pltpu.VMEM((_N, emb), jnp.float32)],
        ),
        out_shape=[jax.ShapeDtypeStruct((_N, emb), jnp.float32),
                   jax.ShapeDtypeStruct((_N, proj), jnp.float32)],
    )(rows, cols, a, z1, z2p, gamma.reshape(1, -1), beta.reshape(1, -1),
      Wp1, bp1.reshape(1, -1), Wp2, bp2.reshape(1, -1))
    return (zn, p)
